# baseline (speedup 1.0000x reference)
import jax
import jax.numpy as jnp
from jax import lax
from jax.experimental import pallas as pl
from jax.experimental.pallas import tpu as pltpu
from jax.experimental.pallas import tpu_sc as plsc

_NUM_CORES = 2
_NUM_SUBCORES = 16
_NW = _NUM_CORES * _NUM_SUBCORES

_B = 4096
_L = 50
_D = 32
_ROWS = _B * _L
_RPW = _ROWS // _NW
_BPW = _B // _NW
_SLEN = 128
_NSTREAM = _RPW // _SLEN
_SPC = 10
_NCHUNK = _NSTREAM // _SPC
_CROWS = _SPC * _SLEN
_MROWS = _ROWS // _SLEN


def _embed_body(table, q1d, lens, out, idx_v, mask_v, lens_v, buf0, buf1,
                g0, g1, o0, o1):
  wid = lax.axis_index("s") * _NUM_CORES + lax.axis_index("c")
  row0 = wid * _RPW

  pltpu.sync_copy(q1d.at[pl.ds(row0, _RPW)], idx_v)

  bufs = (buf0, buf1)
  gsem = (g0, g1)
  osem = (o0, o1)

  def fire_gather(g):
    buf = bufs[g % 2]
    return [
        pltpu.async_copy(table.at[idx_v.at[pl.ds((g * _SPC + s) * _SLEN, _SLEN)]],
                         buf.at[pl.ds(s * _SLEN, _SLEN)], gsem[g % 2])
        for s in range(_SPC)
    ]

  gh = [None] * _NCHUNK
  oh = [None] * _NCHUNK
  gh[0] = fire_gather(0)

  pltpu.sync_copy(lens.at[pl.ds(wid * _BPW, _BPW)], lens_v)

  def mask_body(j, _):
    p = j * 16 + lax.iota(jnp.int32, 16)
    r = lax.shift_right_logical(p * 5243, 18)
    c = p - r * _L
    lv = plsc.load_gather(lens_v, [r])
    mask_v[pl.ds(j * 16, 16)] = jnp.where(c < lv, jnp.float32(1.0),
                                          jnp.float32(0.0))
    return 0

  lax.fori_loop(0, _RPW // 16, mask_body, 0, unroll=4)

  for g in range(_NCHUNK):
    b = g % 2
    buf = bufs[b]
    if g + 1 < _NCHUNK:
      if g >= 1:
        oh[g - 1].wait()
      gh[g + 1] = fire_gather(g + 1)
    for h in gh[g]:
      h.wait()
    base = g * _CROWS

    def mblock(jb, _):
      mv = mask_v[pl.ds(base + jb * 16, 16)]
      rb = jb * 16
      for r in range(16):
        m = mv[r]
        buf[rb + r, pl.ds(0, 16)] = buf[rb + r, pl.ds(0, 16)] * m
        buf[rb + r, pl.ds(16, 16)] = buf[rb + r, pl.ds(16, 16)] * m
      return 0

    lax.fori_loop(0, _CROWS // 16, mblock, 0)
    oh[g] = pltpu.async_copy(buf, out.at[pl.ds(row0 + g * _CROWS, _CROWS)],
                             osem[b])
  oh[_NCHUNK - 2].wait()
  oh[_NCHUNK - 1].wait()


def kernel(queries, query_lens, embedding_weight):
  q1d = queries.astype(jnp.int32).reshape(_ROWS)
  lens = query_lens.astype(jnp.int32)
  mesh = plsc.VectorSubcoreMesh(core_axis_name="c", subcore_axis_name="s",
                                num_cores=_NUM_CORES,
                                num_subcores=_NUM_SUBCORES)
  out = pl.kernel(
      _embed_body,
      out_type=jax.ShapeDtypeStruct((_ROWS, _D), jnp.float32),
      mesh=mesh,
      compiler_params=pltpu.CompilerParams(use_tc_tiling_on_sc=False,
                                           needs_layout_passes=False),
      scratch_types=[
          pltpu.VMEM((_RPW,), jnp.int32),
          pltpu.VMEM((_RPW,), jnp.float32),
          pltpu.VMEM((_BPW,), jnp.int32),
          pltpu.VMEM((_CROWS, _D), jnp.float32),
          pltpu.VMEM((_CROWS, _D), jnp.float32),
          pltpu.SemaphoreType.DMA,
          pltpu.SemaphoreType.DMA,
          pltpu.SemaphoreType.DMA,
          pltpu.SemaphoreType.DMA,
      ],
  )(embedding_weight, q1d, lens)
  return out.reshape(_B, _L, _D)
